# baseline (speedup 1.0000x reference)
import functools
import math

import jax
import jax.numpy as jnp
from jax.experimental import pallas as pl
from jax.experimental.pallas import tpu as pltpu

_B, _T, _D, _H, _N = 8, 64, 1024, 768, 16384
_K = 4096
_NC = 1024
_NCH = _N // _NC
_R = _B * _T


def _normalize_body(cx_ref, cy_ref, cz_ref, oxn_ref, oyn_ref, ozn_ref):
    cx, cy, cz = cx_ref[...], cy_ref[...], cz_ref[...]
    norm = jnp.maximum(jnp.sqrt(cx * cx + cy * cy + cz * cz), 1e-12)
    oxn_ref[...] = cx / norm
    oyn_ref[...] = cy / norm
    ozn_ref[...] = cz / norm


def _decode_body(lt_ref, ls_ref, thr_ref, need_ref, ct_ref, w3t_ref,
                 b3_ref, w4_ref, b4_ref, out_ref, m_ref, d_ref, y_ref,
                 base_ref):
    c = pl.program_id(0)

    @pl.when(c == 0)
    def _init():
        m_ref[...] = jnp.full_like(m_ref, -1e30)
        d_ref[...] = jnp.zeros_like(d_ref)
        y_ref[...] = jnp.zeros_like(y_ref)
        base_ref[...] = jnp.zeros_like(base_ref)

    ls = ls_ref[...]
    thr = thr_ref[...]
    gt = ls > thr
    eq = (ls == thr).astype(jnp.float32)
    lr = eq
    s = 1
    while s < _NC:
        lr = lr + jnp.concatenate(
            [jnp.zeros((_B, s), jnp.float32), lr[:, :-s]], axis=1)
        s *= 2
    rank = base_ref[...] + lr
    sel = jnp.where(gt | ((eq > 0.5) & (rank <= need_ref[...])), 1.0, 0.0)
    base_ref[...] += jnp.sum(eq, axis=1, keepdims=True)

    sel512 = jnp.broadcast_to(sel[:, None, :],
                              (_B, _T, _NC)).reshape(_R, _NC)
    l = jnp.where(sel512 > 0.5, lt_ref[...], -1e30)
    mc = jnp.max(l, axis=1, keepdims=True)
    m_old = m_ref[...]
    mnew = jnp.maximum(m_old, mc)
    scale = jnp.exp(m_old - mnew)
    p = jnp.exp(l - mnew) * sel512
    d_ref[...] = d_ref[...] * scale + jnp.sum(p, axis=1, keepdims=True)

    cx = ct_ref[0:1, :]
    cy = ct_ref[1:2, :]
    cz = ct_ref[2:3, :]
    nrm = jnp.maximum(jnp.sqrt(cx * cx + cy * cy + cz * cz), 1e-12)
    at = (w3t_ref[:, 0:1] * (cx / nrm) + w3t_ref[:, 1:2] * (cy / nrm)
          + w3t_ref[:, 2:3] * (cz / nrm) + b3_ref[...])
    at = jnp.maximum(at, 0.0)
    y_ref[...] = y_ref[...] * scale + jax.lax.dot_general(
        p.astype(jnp.bfloat16), at.astype(jnp.bfloat16),
        dimension_numbers=(((1,), (1,)), ((), ())),
        preferred_element_type=jnp.float32)
    m_ref[...] = mnew

    @pl.when(c == _NCH - 1)
    def _fin():
        yn = (y_ref[...] / d_ref[...]) * math.sqrt(_K)
        out_ref[...] = (jnp.dot(yn.astype(jnp.bfloat16), w4_ref[...],
                                preferred_element_type=jnp.float32)
                        + math.sqrt(_K) * b4_ref[...])


@functools.partial(jax.jit, static_argnames=("interpret",))
def _decode(lt2d, ls, thr, need, compT, W3T, b3c, W4b, b4,
            interpret=False):
    return pl.pallas_call(
        _decode_body,
        grid=(_NCH,),
        in_specs=[
            pl.BlockSpec((_R, _NC), lambda c: (0, c)),
            pl.BlockSpec((_B, _NC), lambda c: (0, c)),
            pl.BlockSpec((_B, 1), lambda c: (0, 0)),
            pl.BlockSpec((_B, 1), lambda c: (0, 0)),
            pl.BlockSpec((3, _NC), lambda c: (0, c)),
            pl.BlockSpec((_H, 3), lambda c: (0, 0)),
            pl.BlockSpec((_H, 1), lambda c: (0, 0)),
            pl.BlockSpec((_H, _D), lambda c: (0, 0)),
            pl.BlockSpec((1, _D), lambda c: (0, 0)),
        ],
        out_specs=pl.BlockSpec((_R, _D), lambda c: (0, 0)),
        out_shape=jax.ShapeDtypeStruct((_R, _D), jnp.float32),
        scratch_shapes=[
            pltpu.VMEM((_R, 1), jnp.float32),
            pltpu.VMEM((_R, 1), jnp.float32),
            pltpu.VMEM((_R, _H), jnp.float32),
            pltpu.VMEM((_B, 1), jnp.float32),
        ],
        interpret=interpret,
    )(lt2d, ls, thr, need, compT, W3T, b3c, W4b, b4)


@functools.partial(jax.jit, static_argnames=("interpret",))
def _normalize(cx, cy, cz, interpret=False):
    return pl.pallas_call(
        _normalize_body,
        out_shape=[jax.ShapeDtypeStruct((_B, _K), jnp.float32)] * 3,
        interpret=interpret,
    )(cx, cy, cz)


def kernel(x, W1, b1, W2, b2, components, W3, b3, W4, b4):
    Bx, Tx, Dx = x.shape
    k = min(64 * Tx, components.shape[0])
    x = x.astype(jnp.float32)
    h = jax.nn.relu(x @ W1 + b1)
    logits_tok = h @ W2 + b2
    logits_sum = logits_tok.sum(axis=1) / math.sqrt(Tx)
    vals, idx = jax.lax.top_k(logits_sum, k)

    thr = vals[:, k - 1:k]
    n_gt = (logits_sum > thr).sum(axis=1, keepdims=True)
    need = (k - n_gt).astype(jnp.float32)

    comps_g = components[idx]
    cxn, cyn, czn = _normalize(comps_g[..., 0], comps_g[..., 1],
                               comps_g[..., 2])
    comps_k = jnp.stack([cxn, cyn, czn], axis=-1)

    xr = _decode(logits_tok.reshape(_R, _N), logits_sum, thr, need,
                 components.T, W3.T, b3.reshape(_H, 1),
                 W4.astype(jnp.bfloat16), b4.reshape(1, _D))
    return xr.reshape(Bx, Tx, Dx), comps_k


# speedup vs baseline: 1.2740x; 1.2740x over previous
import functools
import math

import jax
import jax.numpy as jnp
from jax import lax
from jax.experimental import pallas as pl
from jax.experimental.pallas import tpu as pltpu
from jax.experimental.pallas import tpu_sc as plsc

_B, _T, _D, _H, _N = 8, 64, 1024, 768, 16384
_K = 4096
_NC = 2048
_NCH = _N // _NC
_R = _B * _T


_NW = 32
_PER = (_B * _K) // _NW


def _sc_gather_body(cx_hbm, cy_hbm, cz_hbm, idx_hbm, out_hbm, cx_v, cy_v,
                    cz_v, idx_v, out_v):
    wid = lax.axis_index("s") * 2 + lax.axis_index("c")
    base = wid * _PER
    pltpu.sync_copy(cx_hbm, cx_v)
    pltpu.sync_copy(cy_hbm, cy_v)
    pltpu.sync_copy(cz_hbm, cz_v)
    pltpu.sync_copy(idx_hbm.at[pl.ds(base, _PER)], idx_v)
    lane = lax.iota(jnp.int32, 16) * 3
    for i in range(_PER // 16):
        iv = idx_v[pl.ds(i * 16, 16)]
        pos = lane + (i * 48)
        plsc.store_scatter(out_v, [pos], plsc.load_gather(cx_v, [iv]))
        plsc.store_scatter(out_v, [pos + 1], plsc.load_gather(cy_v, [iv]))
        plsc.store_scatter(out_v, [pos + 2], plsc.load_gather(cz_v, [iv]))
    pltpu.sync_copy(out_v, out_hbm.at[pl.ds(base * 3, _PER * 3)])


@jax.jit
def _sc_gather(cxh, cyh, czh, idx_flat):
    import functools as _ft
    kern = _ft.partial(
        pl.kernel,
        mesh=plsc.VectorSubcoreMesh(core_axis_name="c",
                                    subcore_axis_name="s"),
        out_type=jax.ShapeDtypeStruct((_B * _K * 3,), jnp.float32),
        scratch_types=[
            pltpu.VMEM((_N,), jnp.float32),
            pltpu.VMEM((_N,), jnp.float32),
            pltpu.VMEM((_N,), jnp.float32),
            pltpu.VMEM((_PER,), jnp.int32),
            pltpu.VMEM((_PER * 3,), jnp.float32),
        ],
        compiler_params=pltpu.CompilerParams(needs_layout_passes=False),
    )(_sc_gather_body)
    return kern(cxh, cyh, czh, idx_flat)


def _normalize_body(cx_ref, cy_ref, cz_ref, oxn_ref, oyn_ref, ozn_ref):
    cx, cy, cz = cx_ref[...], cy_ref[...], cz_ref[...]
    norm = jnp.maximum(jnp.sqrt(cx * cx + cy * cy + cz * cz), 1e-12)
    oxn_ref[...] = cx / norm
    oyn_ref[...] = cy / norm
    ozn_ref[...] = cz / norm


def _decode_body(lt_ref, ls_ref, thr_ref, need_ref, ct_ref, w3t_ref,
                 b3_ref, w4_ref, b4_ref, out_ref, m_ref, d_ref, y_ref,
                 base_ref):
    c = pl.program_id(0)

    @pl.when(c == 0)
    def _init():
        m_ref[...] = jnp.full_like(m_ref, -1e30)
        d_ref[...] = jnp.zeros_like(d_ref)
        y_ref[...] = jnp.zeros_like(y_ref)
        base_ref[...] = jnp.zeros_like(base_ref)

    ls = ls_ref[...]
    thr = thr_ref[...]
    gt = ls > thr
    eq = (ls == thr).astype(jnp.float32)
    lr = eq
    s = 1
    while s < _NC:
        lr = lr + jnp.concatenate(
            [jnp.zeros((_B, s), jnp.float32), lr[:, :-s]], axis=1)
        s *= 2
    rank = base_ref[...] + lr
    sel = jnp.where(gt | ((eq > 0.5) & (rank <= need_ref[...])), 1.0, 0.0)
    base_ref[...] += jnp.sum(eq, axis=1, keepdims=True)

    sel512 = jnp.broadcast_to(sel[:, None, :],
                              (_B, _T, _NC)).reshape(_R, _NC)
    l = jnp.where(sel512 > 0.5, lt_ref[...], -1e30)
    mc = jnp.max(l, axis=1, keepdims=True)
    m_old = m_ref[...]
    mnew = jnp.maximum(m_old, mc)
    scale = jnp.exp(m_old - mnew)
    p = jnp.exp(l - mnew) * sel512
    d_ref[...] = d_ref[...] * scale + jnp.sum(p, axis=1, keepdims=True)

    cx = ct_ref[0:1, :]
    cy = ct_ref[1:2, :]
    cz = ct_ref[2:3, :]
    nrm = jnp.maximum(jnp.sqrt(cx * cx + cy * cy + cz * cz), 1e-12)
    at = (w3t_ref[:, 0:1] * (cx / nrm) + w3t_ref[:, 1:2] * (cy / nrm)
          + w3t_ref[:, 2:3] * (cz / nrm) + b3_ref[...])
    at = jnp.maximum(at, 0.0)
    y_ref[...] = y_ref[...] * scale + jax.lax.dot_general(
        p.astype(jnp.bfloat16), at.astype(jnp.bfloat16),
        dimension_numbers=(((1,), (1,)), ((), ())),
        preferred_element_type=jnp.float32)
    m_ref[...] = mnew

    @pl.when(c == _NCH - 1)
    def _fin():
        yn = (y_ref[...] / d_ref[...]) * math.sqrt(_K)
        out_ref[...] = (jnp.dot(yn.astype(jnp.bfloat16), w4_ref[...],
                                preferred_element_type=jnp.float32)
                        + math.sqrt(_K) * b4_ref[...])


@functools.partial(jax.jit, static_argnames=("interpret",))
def _decode(lt2d, ls, thr, need, compT, W3T, b3c, W4b, b4,
            interpret=False):
    return pl.pallas_call(
        _decode_body,
        grid=(_NCH,),
        in_specs=[
            pl.BlockSpec((_R, _NC), lambda c: (0, c)),
            pl.BlockSpec((_B, _NC), lambda c: (0, c)),
            pl.BlockSpec((_B, 1), lambda c: (0, 0)),
            pl.BlockSpec((_B, 1), lambda c: (0, 0)),
            pl.BlockSpec((3, _NC), lambda c: (0, c)),
            pl.BlockSpec((_H, 3), lambda c: (0, 0)),
            pl.BlockSpec((_H, 1), lambda c: (0, 0)),
            pl.BlockSpec((_H, _D), lambda c: (0, 0)),
            pl.BlockSpec((1, _D), lambda c: (0, 0)),
        ],
        out_specs=pl.BlockSpec((_R, _D), lambda c: (0, 0)),
        out_shape=jax.ShapeDtypeStruct((_R, _D), jnp.float32),
        scratch_shapes=[
            pltpu.VMEM((_R, 1), jnp.float32),
            pltpu.VMEM((_R, 1), jnp.float32),
            pltpu.VMEM((_R, _H), jnp.float32),
            pltpu.VMEM((_B, 1), jnp.float32),
        ],
        interpret=interpret,
    )(lt2d, ls, thr, need, compT, W3T, b3c, W4b, b4)


@functools.partial(jax.jit, static_argnames=("interpret",))
def _normalize(cx, cy, cz, interpret=False):
    return pl.pallas_call(
        _normalize_body,
        out_shape=[jax.ShapeDtypeStruct((_B, _K), jnp.float32)] * 3,
        interpret=interpret,
    )(cx, cy, cz)


def kernel(x, W1, b1, W2, b2, components, W3, b3, W4, b4):
    Bx, Tx, Dx = x.shape
    k = min(64 * Tx, components.shape[0])
    x = x.astype(jnp.float32)
    h = jax.nn.relu(x @ W1 + b1)
    logits_tok = h @ W2 + b2
    logits_sum = logits_tok.sum(axis=1) / math.sqrt(Tx)
    vals, idx = jax.lax.top_k(logits_sum, k)

    thr = vals[:, k - 1:k]
    n_gt = (logits_sum > thr).sum(axis=1, keepdims=True)
    need = (k - n_gt).astype(jnp.float32)

    compT = components.T
    comps_g = _sc_gather(compT[0], compT[1], compT[2],
                         idx.reshape(-1)).reshape(Bx, k, 3)
    cxn, cyn, czn = _normalize(comps_g[..., 0], comps_g[..., 1],
                               comps_g[..., 2])
    comps_k = jnp.stack([cxn, cyn, czn], axis=-1)

    xr = _decode(logits_tok.reshape(_R, _N), logits_sum, thr, need,
                 compT, W3.T, b3.reshape(_H, 1),
                 W4.astype(jnp.bfloat16), b4.reshape(1, _D))
    return xr.reshape(Bx, Tx, Dx), comps_k
